# Initial kernel scaffold; baseline (speedup 1.0000x reference)
import jax
import jax.numpy as jnp
from jax.experimental import pallas as pl


def kernel(input, adj, weight1, weight2):
    raise NotImplementedError("write your pallas kernel here")


# speedup vs baseline: 1.0559x; 1.0559x over previous
import jax
import jax.numpy as jnp
from jax.experimental import pallas as pl
from jax.experimental.pallas import tpu as pltpu

_N = 10000
_IN_F = 128
_MID = 32
_OUT_F = 128
_BM = 400


def _gcn_kernel(x_ref, w1_ref, adj_ref, w2_ref, out_ref, hid_ref):
    @pl.when(pl.program_id(0) == 0)
    def _():
        h = jnp.dot(
            x_ref[...].astype(jnp.bfloat16),
            w1_ref[...].astype(jnp.bfloat16),
            preferred_element_type=jnp.float32,
        )
        hid_ref[...] = h.astype(jnp.bfloat16)

    t = jnp.dot(
        adj_ref[...].astype(jnp.bfloat16),
        hid_ref[...],
        preferred_element_type=jnp.float32,
    )
    out_ref[...] = jnp.dot(
        t.astype(jnp.bfloat16),
        w2_ref[...].astype(jnp.bfloat16),
        preferred_element_type=jnp.float32,
    )


def kernel(input, adj, weight1, weight2):
    grid = (_N // _BM,)
    return pl.pallas_call(
        _gcn_kernel,
        grid=grid,
        in_specs=[
            pl.BlockSpec((_N, _IN_F), lambda i: (0, 0)),
            pl.BlockSpec((_IN_F, _MID), lambda i: (0, 0)),
            pl.BlockSpec((_BM, _N), lambda i: (i, 0)),
            pl.BlockSpec((_MID, _OUT_F), lambda i: (0, 0)),
        ],
        out_specs=pl.BlockSpec((_BM, _OUT_F), lambda i: (i, 0)),
        out_shape=jax.ShapeDtypeStruct((_N, _OUT_F), jnp.float32),
        scratch_shapes=[pltpu.VMEM((_N, _MID), jnp.bfloat16)],
        compiler_params=pltpu.CompilerParams(
            dimension_semantics=("arbitrary",),
        ),
    )(input, weight1, adj, weight2)
